# baseline (speedup 1.0000x reference)
import functools

import jax
import jax.numpy as jnp
from jax import lax
from jax.experimental import pallas as pl
from jax.experimental.pallas import tpu as pltpu
from jax.experimental.pallas import tpu_sc as plsc

_NC = 2
_NS = 16
_NW = _NC * _NS
_LANES = 16
_UNROLL = 4


def _chunk_words(total_words: int) -> int:
    per = -(-total_words // _NW)
    q = _LANES * _UNROLL
    return -(-per // q) * q


@functools.lru_cache(maxsize=None)
def _sc_mask_kernel(chunk: int):
    total = chunk * _NW
    mesh = plsc.VectorSubcoreMesh(core_axis_name="c", subcore_axis_name="s")

    @functools.partial(
        pl.kernel,
        mesh=mesh,
        out_type=jax.ShapeDtypeStruct((total,), jnp.int32),
        scratch_types=[
            pltpu.VMEM((chunk,), jnp.int32),
            pltpu.VMEM((chunk,), jnp.int32),
        ],
    )
    def body(m_hbm, t_hbm, out_hbm, m_v, t_v):
        wid = lax.axis_index("s") * _NC + lax.axis_index("c")
        base = wid * chunk
        pltpu.sync_copy(m_hbm.at[pl.ds(base, chunk)], m_v)
        pltpu.sync_copy(t_hbm.at[pl.ds(base, chunk)], t_v)

        def step(i, carry):
            b = i * (_LANES * _UNROLL)
            for u in range(_UNROLL):
                sl = pl.ds(b + u * _LANES, _LANES)
                m_v[sl] = m_v[sl] & ~t_v[sl]
            return carry

        lax.fori_loop(0, chunk // (_LANES * _UNROLL), step, 0)
        pltpu.sync_copy(m_v, out_hbm.at[pl.ds(base, chunk)])

    return body


def kernel(s0, s1, s2, mask, track_mask):
    n = mask.shape[0]
    chunk = _chunk_words(-(-n // 4))
    total = chunk * _NW
    nbytes = total * 4

    m8 = jnp.pad(mask.astype(jnp.uint8), (0, nbytes - n))
    t8 = jnp.pad(track_mask.astype(jnp.uint8), (0, nbytes - n))
    m32 = lax.bitcast_convert_type(m8.reshape(total, 4), jnp.int32)
    t32 = lax.bitcast_convert_type(t8.reshape(total, 4), jnp.int32)

    out32 = _sc_mask_kernel(chunk)(m32, t32)

    out8 = lax.bitcast_convert_type(out32, jnp.uint8).reshape(nbytes)[:n]
    new_mask = out8.astype(jnp.bool_)
    return (s0, s1, s2, new_mask)


# speedup vs baseline: 8.6734x; 8.6734x over previous
import functools

import jax
import jax.numpy as jnp
from jax import lax
from jax.experimental import pallas as pl
from jax.experimental.pallas import tpu as pltpu
from jax.experimental.pallas import tpu_sc as plsc

_NC = 2
_NS = 16
_NW = _NC * _NS
_LANES = 16
_UNROLL = 4


def _chunk_words(total_words: int) -> int:
    per = -(-total_words // _NW)
    q = 512
    return -(-per // q) * q


@functools.lru_cache(maxsize=None)
def _sc_mask_kernel(chunk: int):
    total = chunk * _NW
    mesh = plsc.VectorSubcoreMesh(core_axis_name="c", subcore_axis_name="s")

    @functools.partial(
        pl.kernel,
        mesh=mesh,
        out_type=jax.ShapeDtypeStruct((total,), jnp.int32),
        scratch_types=[
            pltpu.VMEM((chunk,), jnp.int32),
            pltpu.VMEM((chunk,), jnp.int32),
        ],
    )
    def body(m_hbm, t_hbm, out_hbm, m_v, t_v):
        wid = lax.axis_index("s") * _NC + lax.axis_index("c")
        base = wid * chunk
        pltpu.sync_copy(m_hbm.at[pl.ds(base, chunk)], m_v)
        pltpu.sync_copy(t_hbm.at[pl.ds(base, chunk)], t_v)

        def step(i, carry):
            b = i * (_LANES * _UNROLL)
            for u in range(_UNROLL):
                sl = pl.ds(b + u * _LANES, _LANES)
                m_v[sl] = m_v[sl] & ~t_v[sl]
            return carry

        lax.fori_loop(0, chunk // (_LANES * _UNROLL), step, 0)
        pltpu.sync_copy(m_v, out_hbm.at[pl.ds(base, chunk)])

    return body


def kernel(s0, s1, s2, mask, track_mask):
    n = mask.shape[0]
    chunk = _chunk_words(n)
    total = chunk * _NW

    m32 = jnp.pad(mask.astype(jnp.int32), (0, total - n))
    t32 = jnp.pad(track_mask.astype(jnp.int32), (0, total - n))

    out32 = _sc_mask_kernel(chunk)(m32, t32)

    new_mask = out32[:n].astype(jnp.bool_)
    return (s0, s1, s2, new_mask)


# speedup vs baseline: 9.4979x; 1.0951x over previous
import functools

import jax
import jax.numpy as jnp
from jax import lax
from jax.experimental import pallas as pl
from jax.experimental.pallas import tpu as pltpu
from jax.experimental.pallas import tpu_sc as plsc

_NC = 2
_NS = 16
_NW = _NC * _NS
_LANES = 16
_MINOR = 128
_ROWQ = 32


def _rows_per_worker(total_bytes: int) -> int:
    per = -(-total_bytes // (_NW * _MINOR))
    return -(-per // _ROWQ) * _ROWQ


@functools.lru_cache(maxsize=None)
def _sc_mask_kernel(rows: int):
    total_rows = rows * _NW
    mesh = plsc.VectorSubcoreMesh(core_axis_name="c", subcore_axis_name="s")

    @functools.partial(
        pl.kernel,
        mesh=mesh,
        out_type=jax.ShapeDtypeStruct((total_rows, _MINOR), jnp.uint8),
        scratch_types=[
            pltpu.VMEM((rows, _MINOR), jnp.uint8),
            pltpu.VMEM((rows, _MINOR), jnp.uint8),
        ],
    )
    def body(m_hbm, t_hbm, out_hbm, m_v, t_v):
        wid = lax.axis_index("s") * _NC + lax.axis_index("c")
        base = wid * rows
        pltpu.sync_copy(m_hbm.at[pl.ds(base, rows)], m_v)
        pltpu.sync_copy(t_hbm.at[pl.ds(base, rows)], t_v)

        mw = m_v.bitcast(jnp.int32)
        tw = t_v.bitcast(jnp.int32)

        def step(r, carry):
            for c in range(_MINOR // _LANES):
                sl = pl.ds(c * _LANES, _LANES)
                mw[r, sl] = mw[r, sl] & ~tw[r, sl]
            return carry

        lax.fori_loop(0, rows // 4, step, 0)
        pltpu.sync_copy(m_v, out_hbm.at[pl.ds(base, rows)])

    return body


def kernel(s0, s1, s2, mask, track_mask):
    n = mask.shape[0]
    rows = _rows_per_worker(n)
    total = rows * _NW * _MINOR

    m = jnp.pad(mask.view(jnp.uint8), (0, total - n)).reshape(rows * _NW, _MINOR)
    t = jnp.pad(track_mask.view(jnp.uint8), (0, total - n)).reshape(rows * _NW, _MINOR)
    out = _sc_mask_kernel(rows)(m, t)
    return (s0, s1, s2, out.reshape(total)[:n].view(jnp.bool_))
